# baseline (speedup 1.0000x reference)
import functools

import jax
import jax.numpy as jnp
from jax import lax
from jax.experimental import pallas as pl
from jax.experimental.pallas import tpu as pltpu
from jax.experimental.pallas import tpu_sc as plsc

DIM = 1024
E = 16
TOPK = 2
N_GROUPS = 4
GROUP_SZ = E // N_GROUPS
TOPK_GROUPS = 2
MOE_INTER = 1024
ROUTE_SCALE = 2.5
T = 2048
CAPACITY = 512

BUF_ROWS = 8192 + 128
TRASH_BASE = 8192

SC_CORES = 2
SC_SUBCORES = 16
SC_WORKERS = SC_CORES * SC_SUBCORES
TOK_PER_W = T // SC_WORKERS


def _gate_body(x_ref, gw_ref, da_ref, ca_ref, wv_ref, cnt_ref):
    x = x_ref[...]
    gw = gw_ref[...]
    logits = lax.dot_general(x, gw, (((1,), (1,)), ((), ())),
                             preferred_element_type=jnp.float32)
    scores = jax.nn.sigmoid(logits)

    lane = lax.broadcasted_iota(jnp.int32, (T, E), 1)
    group_of_lane = lane // GROUP_SZ

    gmax = []
    for g in range(N_GROUPS):
        m = jnp.max(jnp.where(group_of_lane == g, scores, -jnp.inf),
                    axis=1, keepdims=True)
        gmax.append(m)
    sel = []
    for g in range(N_GROUPS):
        rank = jnp.zeros_like(gmax[g])
        for h in range(N_GROUPS):
            if h == g:
                continue
            beats = (gmax[h] > gmax[g]) | ((gmax[h] == gmax[g]) & (h < g))
            rank = rank + beats.astype(jnp.float32)
        sel.append(rank < TOPK_GROUPS)
    group_mask = jnp.zeros((T, E), dtype=jnp.bool_)
    for g in range(N_GROUPS):
        group_mask = group_mask | (sel[g] & (group_of_lane == g))

    ms = jnp.where(group_mask, scores, 0.0)

    BIG = jnp.int32(999)
    v1 = jnp.max(ms, axis=1, keepdims=True)
    i1 = jnp.min(jnp.where(ms == v1, lane, BIG), axis=1, keepdims=True)
    ms2 = jnp.where(lane == i1, -1.0, ms)
    v2 = jnp.max(ms2, axis=1, keepdims=True)
    i2 = jnp.min(jnp.where(ms2 == v2, lane, BIG), axis=1, keepdims=True)

    w1 = jnp.sum(jnp.where(lane == i1, scores, 0.0), axis=1, keepdims=True)
    w2 = jnp.sum(jnp.where(lane == i2, scores, 0.0), axis=1, keepdims=True)
    denom = w1 + w2
    w1n = w1 / denom * ROUTE_SCALE
    w2n = w2 / denom * ROUTE_SCALE

    oh1 = (lane == i1).astype(jnp.float32)
    oh2 = (lane == i2).astype(jnp.float32)
    oh = oh1 + oh2
    CH = 256
    r = lax.broadcasted_iota(jnp.int32, (CH, CH), 0)
    c = lax.broadcasted_iota(jnp.int32, (CH, CH), 1)
    ltri = (c < r).astype(jnp.bfloat16)
    ohb = oh.astype(jnp.bfloat16)
    carry = jnp.zeros((1, E), jnp.float32)
    chunks = []
    for b in range(T // CH):
        chunk = ohb[b * CH:(b + 1) * CH, :]
        chunks.append(jnp.dot(ltri, chunk,
                              preferred_element_type=jnp.float32) + carry)
        carry = carry + jnp.sum(chunk.astype(jnp.float32), axis=0,
                                keepdims=True)
    cum_excl = jnp.concatenate(chunks, axis=0)

    pos1 = jnp.sum(jnp.where(lane == i1, cum_excl, 0.0), axis=1, keepdims=True)
    pos2 = jnp.sum(jnp.where(lane == i2, cum_excl, 0.0), axis=1, keepdims=True)
    pos1 = pos1.astype(jnp.int32)
    pos2 = pos2.astype(jnp.int32)

    valid1 = pos1 < CAPACITY
    valid2 = pos2 < CAPACITY
    pc1 = jnp.minimum(pos1, CAPACITY - 1)
    pc2 = jnp.minimum(pos2, CAPACITY - 1)
    ca1 = i1 * CAPACITY + pc1
    ca2 = i2 * CAPACITY + pc2

    trow = lax.broadcasted_iota(jnp.int32, (T, 1), 0)
    trash = TRASH_BASE + (trow & 7)
    da1 = jnp.where(valid1, ca1, trash)
    da2 = jnp.where(valid2, ca2, trash)

    da_ref[...] = jnp.concatenate([da1, da2], axis=1)
    ca_ref[...] = jnp.concatenate([ca1, ca2], axis=1)
    wv_ref[...] = jnp.concatenate(
        [jnp.where(valid1, w1n, 0.0), jnp.where(valid2, w2n, 0.0)], axis=1)
    cnt_ref[...] = jnp.sum(oh, axis=0, keepdims=True).astype(jnp.int32)


def _gate_call(x, gate_w):
    return pl.pallas_call(
        _gate_body,
        out_shape=[
            jax.ShapeDtypeStruct((T, TOPK), jnp.int32),
            jax.ShapeDtypeStruct((T, TOPK), jnp.int32),
            jax.ShapeDtypeStruct((T, TOPK), jnp.float32),
            jax.ShapeDtypeStruct((1, E), jnp.int32),
        ],
        compiler_params=pltpu.CompilerParams(
            vmem_limit_bytes=60 * 1024 * 1024),
    )(x, gate_w)


HC = TOK_PER_W // 2


def _dispatch_sc(x, da1, da2):
    mesh = plsc.VectorSubcoreMesh(core_axis_name="c", subcore_axis_name="s")

    @functools.partial(
        pl.kernel,
        out_type=jax.ShapeDtypeStruct((BUF_ROWS, DIM), jnp.float32),
        mesh=mesh,
        scratch_types=[
            pltpu.VMEM((HC,), jnp.int32),
            pltpu.VMEM((HC,), jnp.int32),
            pltpu.VMEM((HC,), jnp.int32),
            pltpu.VMEM((HC,), jnp.int32),
            pltpu.VMEM((HC, DIM), jnp.float32),
            pltpu.VMEM((HC, DIM), jnp.float32),
            pltpu.SemaphoreType.DMA,
            pltpu.SemaphoreType.DMA,
            pltpu.SemaphoreType.DMA,
        ],
    )
    def k(x_hbm, da1_hbm, da2_hbm, buf_hbm,
          i1a, i1b, i2a, i2b, ra, rb, sa, sb, ss):
        wid = lax.axis_index("s") * SC_CORES + lax.axis_index("c")
        base = wid * TOK_PER_W
        la = pltpu.async_copy(x_hbm.at[pl.ds(base, HC)], ra, sa)
        lb = pltpu.async_copy(x_hbm.at[pl.ds(base + HC, HC)], rb, sb)
        pltpu.sync_copy(da1_hbm.at[pl.ds(base, HC)], i1a)
        pltpu.sync_copy(da1_hbm.at[pl.ds(base + HC, HC)], i1b)
        pltpu.sync_copy(da2_hbm.at[pl.ds(base, HC)], i2a)
        pltpu.sync_copy(da2_hbm.at[pl.ds(base + HC, HC)], i2b)
        la.wait()
        s1 = pltpu.async_copy(ra, buf_hbm.at[i1a], ss)
        s2 = pltpu.async_copy(ra, buf_hbm.at[i2a], ss)
        lb.wait()
        s3 = pltpu.async_copy(rb, buf_hbm.at[i1b], ss)
        s4 = pltpu.async_copy(rb, buf_hbm.at[i2b], ss)
        s1.wait()
        s2.wait()
        s3.wait()
        s4.wait()

    return k(x, da1, da2)


def _shared_body(x_ref, ws1_ref, ws3_ref, ws2_ref, z_ref):
    xb = x_ref[...].astype(jnp.bfloat16)
    nt = (((1,), (1,)), ((), ()))
    h = lax.dot_general(xb, ws1_ref[...].astype(jnp.bfloat16), nt,
                        preferred_element_type=jnp.float32)
    g = lax.dot_general(xb, ws3_ref[...].astype(jnp.bfloat16), nt,
                        preferred_element_type=jnp.float32)
    a = (jax.nn.silu(h) * g).astype(jnp.bfloat16)
    z_ref[...] = lax.dot_general(a, ws2_ref[...].astype(jnp.bfloat16), nt,
                                 preferred_element_type=jnp.float32)


def _shared_call(x, ws1, ws3, ws2):
    BT = 512
    return pl.pallas_call(
        _shared_body,
        grid=(T // BT,),
        in_specs=[
            pl.BlockSpec((BT, DIM), lambda i: (i, 0)),
            pl.BlockSpec(ws1.shape, lambda i: (0, 0)),
            pl.BlockSpec(ws3.shape, lambda i: (0, 0)),
            pl.BlockSpec(ws2.shape, lambda i: (0, 0)),
        ],
        out_specs=pl.BlockSpec((BT, DIM), lambda i: (i, 0)),
        out_shape=jax.ShapeDtypeStruct((T, DIM), jnp.float32),
        compiler_params=pltpu.CompilerParams(
            vmem_limit_bytes=60 * 1024 * 1024),
    )(x, ws1, ws3, ws2)


FFN_BM = 64


def _ffn_body(cnt_ref, buf_ref, w1_ref, w3_ref, w2_ref, out_ref):
    e = pl.program_id(0)
    c = jnp.minimum(cnt_ref[0, e], CAPACITY)
    nb = (c + FFN_BM - 1) // FFN_BM

    @pl.when(nb > 0)
    def _():
        w1b = w1_ref[0].astype(jnp.bfloat16)
        w3b = w3_ref[0].astype(jnp.bfloat16)
        w2b = w2_ref[0].astype(jnp.bfloat16)
        nt = (((1,), (1,)), ((), ()))

        def make_branch(m_rows):
            def br():
                xb = buf_ref[0:m_rows, :].astype(jnp.bfloat16)
                h = lax.dot_general(xb, w1b, nt,
                                    preferred_element_type=jnp.float32)
                g = lax.dot_general(xb, w3b, nt,
                                    preferred_element_type=jnp.float32)
                a = (jax.nn.silu(h) * g).astype(jnp.bfloat16)
                out_ref[0:m_rows, :] = lax.dot_general(
                    a, w2b, nt, preferred_element_type=jnp.float32)
            return br

        lax.switch(nb - 1,
                   [make_branch(m * FFN_BM)
                    for m in range(1, CAPACITY // FFN_BM + 1)])


def _ffn_call(counts, buf, w1, w3, w2):
    grid_spec = pltpu.PrefetchScalarGridSpec(
        num_scalar_prefetch=1,
        grid=(E,),
        in_specs=[
            pl.BlockSpec((CAPACITY, DIM), lambda e, cnt: (e, 0)),
            pl.BlockSpec((1, MOE_INTER, DIM), lambda e, cnt: (e, 0, 0)),
            pl.BlockSpec((1, MOE_INTER, DIM), lambda e, cnt: (e, 0, 0)),
            pl.BlockSpec((1, DIM, MOE_INTER), lambda e, cnt: (e, 0, 0)),
        ],
        out_specs=pl.BlockSpec((CAPACITY, DIM), lambda e, cnt: (e, 0)),
    )
    return pl.pallas_call(
        _ffn_body,
        grid_spec=grid_spec,
        out_shape=jax.ShapeDtypeStruct((E * CAPACITY, DIM), jnp.float32),
        compiler_params=pltpu.CompilerParams(
            vmem_limit_bytes=60 * 1024 * 1024,
            dimension_semantics=("arbitrary",)),
    )(counts, buf, w1, w3, w2)


def _gather_sc(out_buf, ca1, ca2):
    mesh = plsc.VectorSubcoreMesh(core_axis_name="c", subcore_axis_name="s")

    @functools.partial(
        pl.kernel,
        out_type=[
            jax.ShapeDtypeStruct((T, DIM), jnp.float32),
            jax.ShapeDtypeStruct((T, DIM), jnp.float32),
        ],
        mesh=mesh,
        scratch_types=[
            pltpu.VMEM((HC,), jnp.int32),
            pltpu.VMEM((HC,), jnp.int32),
            pltpu.VMEM((HC,), jnp.int32),
            pltpu.VMEM((HC,), jnp.int32),
            pltpu.VMEM((HC, DIM), jnp.float32),
            pltpu.VMEM((HC, DIM), jnp.float32),
            pltpu.SemaphoreType.DMA,
            pltpu.SemaphoreType.DMA,
        ],
    )
    def k(ob_hbm, ca1_hbm, ca2_hbm, g1_hbm, g2_hbm,
          iA, iB, iC, iD, ra, rb, sg, sw):
        wid = lax.axis_index("s") * SC_CORES + lax.axis_index("c")
        base = wid * TOK_PER_W
        pltpu.sync_copy(ca1_hbm.at[pl.ds(base, HC)], iA)
        pltpu.sync_copy(ca1_hbm.at[pl.ds(base + HC, HC)], iB)
        pltpu.sync_copy(ca2_hbm.at[pl.ds(base, HC)], iC)
        pltpu.sync_copy(ca2_hbm.at[pl.ds(base + HC, HC)], iD)
        gA = pltpu.async_copy(ob_hbm.at[iA], ra, sg)
        gC = pltpu.async_copy(ob_hbm.at[iC], rb, sg)
        gA.wait()
        wA = pltpu.async_copy(ra, g1_hbm.at[pl.ds(base, HC)], sw)
        gC.wait()
        wC = pltpu.async_copy(rb, g2_hbm.at[pl.ds(base, HC)], sw)
        wA.wait()
        gB = pltpu.async_copy(ob_hbm.at[iB], ra, sg)
        wC.wait()
        gD = pltpu.async_copy(ob_hbm.at[iD], rb, sg)
        gB.wait()
        wB = pltpu.async_copy(ra, g1_hbm.at[pl.ds(base + HC, HC)], sw)
        gD.wait()
        wD = pltpu.async_copy(rb, g2_hbm.at[pl.ds(base + HC, HC)], sw)
        wB.wait()
        wD.wait()

    return k(out_buf, ca1, ca2)


def _combine_body(z_ref, g1_ref, g2_ref, wv_ref, y_ref):
    wv = wv_ref[...]
    y_ref[...] = (z_ref[...]
                  + wv[:, 0:1] * g1_ref[...]
                  + wv[:, 1:2] * g2_ref[...])


def _combine_call(z, g1, g2, wv):
    BT = 512
    return pl.pallas_call(
        _combine_body,
        grid=(T // BT,),
        in_specs=[
            pl.BlockSpec((BT, DIM), lambda i: (i, 0)),
            pl.BlockSpec((BT, DIM), lambda i: (i, 0)),
            pl.BlockSpec((BT, DIM), lambda i: (i, 0)),
            pl.BlockSpec((BT, TOPK), lambda i: (i, 0)),
        ],
        out_specs=pl.BlockSpec((BT, DIM), lambda i: (i, 0)),
        out_shape=jax.ShapeDtypeStruct((T, DIM), jnp.float32),
        compiler_params=pltpu.CompilerParams(
            vmem_limit_bytes=60 * 1024 * 1024),
    )(z, g1, g2, wv)


def kernel(x, gate_w, w1, w2, w3, ws1, ws2, ws3):
    da, ca, wv, counts = _gate_call(x, gate_w)
    buf = _dispatch_sc(x, da[:, 0], da[:, 1])
    z = _shared_call(x, ws1, ws3, ws2)
    out_buf = _ffn_call(counts, buf, w1, w3, w2)
    g1, g2 = _gather_sc(out_buf, ca[:, 0], ca[:, 1])
    return _combine_call(z, g1, g2, wv)


# speedup vs baseline: 1.0375x; 1.0375x over previous
import functools

import jax
import jax.numpy as jnp
from jax import lax
from jax.experimental import pallas as pl
from jax.experimental.pallas import tpu as pltpu
from jax.experimental.pallas import tpu_sc as plsc

DIM = 1024
E = 16
TOPK = 2
N_GROUPS = 4
GROUP_SZ = E // N_GROUPS
TOPK_GROUPS = 2
MOE_INTER = 1024
ROUTE_SCALE = 2.5
T = 2048
CAPACITY = 512

BUF_ROWS = 8192 + 128
TRASH_BASE = 8192

SC_CORES = 2
SC_SUBCORES = 16
SC_WORKERS = SC_CORES * SC_SUBCORES
TOK_PER_W = T // SC_WORKERS


def _gate_body(x_ref, gw_ref, da_ref, ca_ref, wv_ref, cnt_ref):
    x = x_ref[...]
    gw = gw_ref[...]
    logits = lax.dot_general(x, gw, (((1,), (1,)), ((), ())),
                             preferred_element_type=jnp.float32)
    scores = jax.nn.sigmoid(logits)

    lane = lax.broadcasted_iota(jnp.int32, (T, E), 1)
    group_of_lane = lane // GROUP_SZ

    gmax = []
    for g in range(N_GROUPS):
        m = jnp.max(jnp.where(group_of_lane == g, scores, -jnp.inf),
                    axis=1, keepdims=True)
        gmax.append(m)
    sel = []
    for g in range(N_GROUPS):
        rank = jnp.zeros_like(gmax[g])
        for h in range(N_GROUPS):
            if h == g:
                continue
            beats = (gmax[h] > gmax[g]) | ((gmax[h] == gmax[g]) & (h < g))
            rank = rank + beats.astype(jnp.float32)
        sel.append(rank < TOPK_GROUPS)
    group_mask = jnp.zeros((T, E), dtype=jnp.bool_)
    for g in range(N_GROUPS):
        group_mask = group_mask | (sel[g] & (group_of_lane == g))

    ms = jnp.where(group_mask, scores, 0.0)

    BIG = jnp.int32(999)
    v1 = jnp.max(ms, axis=1, keepdims=True)
    i1 = jnp.min(jnp.where(ms == v1, lane, BIG), axis=1, keepdims=True)
    ms2 = jnp.where(lane == i1, -1.0, ms)
    v2 = jnp.max(ms2, axis=1, keepdims=True)
    i2 = jnp.min(jnp.where(ms2 == v2, lane, BIG), axis=1, keepdims=True)

    w1 = jnp.sum(jnp.where(lane == i1, scores, 0.0), axis=1, keepdims=True)
    w2 = jnp.sum(jnp.where(lane == i2, scores, 0.0), axis=1, keepdims=True)
    denom = w1 + w2
    w1n = w1 / denom * ROUTE_SCALE
    w2n = w2 / denom * ROUTE_SCALE

    oh1 = (lane == i1).astype(jnp.float32)
    oh2 = (lane == i2).astype(jnp.float32)
    oh = oh1 + oh2
    CH = 256
    r = lax.broadcasted_iota(jnp.int32, (CH, CH), 0)
    c = lax.broadcasted_iota(jnp.int32, (CH, CH), 1)
    ltri = (c < r).astype(jnp.bfloat16)
    ohb = oh.astype(jnp.bfloat16)
    carry = jnp.zeros((1, E), jnp.float32)
    chunks = []
    for b in range(T // CH):
        chunk = ohb[b * CH:(b + 1) * CH, :]
        chunks.append(jnp.dot(ltri, chunk,
                              preferred_element_type=jnp.float32) + carry)
        carry = carry + jnp.sum(chunk.astype(jnp.float32), axis=0,
                                keepdims=True)
    cum_excl = jnp.concatenate(chunks, axis=0)

    pos1 = jnp.sum(jnp.where(lane == i1, cum_excl, 0.0), axis=1, keepdims=True)
    pos2 = jnp.sum(jnp.where(lane == i2, cum_excl, 0.0), axis=1, keepdims=True)
    pos1 = pos1.astype(jnp.int32)
    pos2 = pos2.astype(jnp.int32)

    valid1 = pos1 < CAPACITY
    valid2 = pos2 < CAPACITY
    pc1 = jnp.minimum(pos1, CAPACITY - 1)
    pc2 = jnp.minimum(pos2, CAPACITY - 1)
    ca1 = i1 * CAPACITY + pc1
    ca2 = i2 * CAPACITY + pc2

    trow = lax.broadcasted_iota(jnp.int32, (T, 1), 0)
    trash = TRASH_BASE + (trow & 7)
    da1 = jnp.where(valid1, ca1, trash)
    da2 = jnp.where(valid2, ca2, trash)

    da_ref[...] = jnp.concatenate([da1, da2], axis=1)
    ca_ref[...] = jnp.concatenate([ca1, ca2], axis=1)
    wv_ref[...] = jnp.concatenate(
        [jnp.where(valid1, w1n, 0.0), jnp.where(valid2, w2n, 0.0)], axis=1)
    cnt_ref[...] = jnp.sum(oh, axis=0, keepdims=True).astype(jnp.int32)


def _gate_call(x, gate_w):
    return pl.pallas_call(
        _gate_body,
        out_shape=[
            jax.ShapeDtypeStruct((T, TOPK), jnp.int32),
            jax.ShapeDtypeStruct((T, TOPK), jnp.int32),
            jax.ShapeDtypeStruct((T, TOPK), jnp.float32),
            jax.ShapeDtypeStruct((1, E), jnp.int32),
        ],
        compiler_params=pltpu.CompilerParams(
            vmem_limit_bytes=60 * 1024 * 1024),
    )(x, gate_w)


HC = TOK_PER_W // 2


def _dispatch_sc(x, da1, da2):
    mesh = plsc.VectorSubcoreMesh(core_axis_name="c", subcore_axis_name="s")

    @functools.partial(
        pl.kernel,
        out_type=jax.ShapeDtypeStruct((BUF_ROWS, DIM), jnp.float32),
        mesh=mesh,
        scratch_types=[
            pltpu.VMEM((HC,), jnp.int32),
            pltpu.VMEM((HC,), jnp.int32),
            pltpu.VMEM((HC,), jnp.int32),
            pltpu.VMEM((HC,), jnp.int32),
            pltpu.VMEM((HC, DIM), jnp.float32),
            pltpu.VMEM((HC, DIM), jnp.float32),
            pltpu.SemaphoreType.DMA,
            pltpu.SemaphoreType.DMA,
            pltpu.SemaphoreType.DMA,
        ],
    )
    def k(x_hbm, da1_hbm, da2_hbm, buf_hbm,
          i1a, i1b, i2a, i2b, ra, rb, sa, sb, ss):
        wid = lax.axis_index("s") * SC_CORES + lax.axis_index("c")
        base = wid * TOK_PER_W
        la = pltpu.async_copy(x_hbm.at[pl.ds(base, HC)], ra, sa)
        lb = pltpu.async_copy(x_hbm.at[pl.ds(base + HC, HC)], rb, sb)
        pltpu.sync_copy(da1_hbm.at[pl.ds(base, HC)], i1a)
        pltpu.sync_copy(da1_hbm.at[pl.ds(base + HC, HC)], i1b)
        pltpu.sync_copy(da2_hbm.at[pl.ds(base, HC)], i2a)
        pltpu.sync_copy(da2_hbm.at[pl.ds(base + HC, HC)], i2b)
        la.wait()
        s1 = pltpu.async_copy(ra, buf_hbm.at[i1a], ss)
        s2 = pltpu.async_copy(ra, buf_hbm.at[i2a], ss)
        lb.wait()
        s3 = pltpu.async_copy(rb, buf_hbm.at[i1b], ss)
        s4 = pltpu.async_copy(rb, buf_hbm.at[i2b], ss)
        s1.wait()
        s2.wait()
        s3.wait()
        s4.wait()

    return k(x, da1, da2)


def _shared_body(x_ref, ws1_ref, ws3_ref, ws2_ref, z_ref):
    xb = x_ref[...].astype(jnp.bfloat16)
    nt = (((1,), (1,)), ((), ()))
    h = lax.dot_general(xb, ws1_ref[...].astype(jnp.bfloat16), nt,
                        preferred_element_type=jnp.float32)
    g = lax.dot_general(xb, ws3_ref[...].astype(jnp.bfloat16), nt,
                        preferred_element_type=jnp.float32)
    a = (jax.nn.silu(h) * g).astype(jnp.bfloat16)
    z_ref[...] = lax.dot_general(a, ws2_ref[...].astype(jnp.bfloat16), nt,
                                 preferred_element_type=jnp.float32)


def _shared_half_a(x, ws1, ws3, ws2):
    BT = 512
    return pl.pallas_call(
        _shared_body,
        grid=(T // (2 * BT),),
        in_specs=[
            pl.BlockSpec((BT, DIM), lambda i: (i, 0)),
            pl.BlockSpec(ws1.shape, lambda i: (0, 0)),
            pl.BlockSpec(ws3.shape, lambda i: (0, 0)),
            pl.BlockSpec(ws2.shape, lambda i: (0, 0)),
        ],
        out_specs=pl.BlockSpec((BT, DIM), lambda i: (i, 0)),
        out_shape=jax.ShapeDtypeStruct((T, DIM), jnp.float32),
        compiler_params=pltpu.CompilerParams(
            vmem_limit_bytes=60 * 1024 * 1024),
    )(x, ws1, ws3, ws2)


def _shared_body_b(z_in_ref, x_ref, ws1_ref, ws3_ref, ws2_ref, dep_ref,
                   z_ref):
    del z_in_ref, dep_ref
    _shared_body(x_ref, ws1_ref, ws3_ref, ws2_ref, z_ref)


def _shared_half_b(z_a, x, ws1, ws3, ws2, dep):
    BT = 512
    H = T // (2 * BT)
    return pl.pallas_call(
        _shared_body_b,
        grid=(H,),
        in_specs=[
            pl.BlockSpec((8, 128), lambda i: (0, 0)),
            pl.BlockSpec((BT, DIM), lambda i: (i + H, 0)),
            pl.BlockSpec(ws1.shape, lambda i: (0, 0)),
            pl.BlockSpec(ws3.shape, lambda i: (0, 0)),
            pl.BlockSpec(ws2.shape, lambda i: (0, 0)),
            pl.BlockSpec((8, 128), lambda i: (0, 0)),
        ],
        out_specs=pl.BlockSpec((BT, DIM), lambda i: (i + H, 0)),
        out_shape=jax.ShapeDtypeStruct((T, DIM), jnp.float32),
        input_output_aliases={0: 0},
        compiler_params=pltpu.CompilerParams(
            vmem_limit_bytes=60 * 1024 * 1024),
    )(z_a, x, ws1, ws3, ws2, dep)


FFN_BM = 64


def _ffn_body(cnt_ref, buf_ref, w1_ref, w3_ref, w2_ref, dep_ref, out_ref):
    del dep_ref
    e = pl.program_id(0)
    c = jnp.minimum(cnt_ref[0, e], CAPACITY)
    nb = (c + FFN_BM - 1) // FFN_BM

    @pl.when(nb > 0)
    def _():
        w1b = w1_ref[0].astype(jnp.bfloat16)
        w3b = w3_ref[0].astype(jnp.bfloat16)
        w2b = w2_ref[0].astype(jnp.bfloat16)
        nt = (((1,), (1,)), ((), ()))

        def make_branch(m_rows):
            def br():
                xb = buf_ref[0:m_rows, :].astype(jnp.bfloat16)
                h = lax.dot_general(xb, w1b, nt,
                                    preferred_element_type=jnp.float32)
                g = lax.dot_general(xb, w3b, nt,
                                    preferred_element_type=jnp.float32)
                a = (jax.nn.silu(h) * g).astype(jnp.bfloat16)
                out_ref[0:m_rows, :] = lax.dot_general(
                    a, w2b, nt, preferred_element_type=jnp.float32)
            return br

        lax.switch(nb - 1,
                   [make_branch(m * FFN_BM)
                    for m in range(1, CAPACITY // FFN_BM + 1)])


def _ffn_call(counts, buf, w1, w3, w2, dep):
    grid_spec = pltpu.PrefetchScalarGridSpec(
        num_scalar_prefetch=1,
        grid=(E,),
        in_specs=[
            pl.BlockSpec((CAPACITY, DIM), lambda e, cnt: (e, 0)),
            pl.BlockSpec((1, MOE_INTER, DIM), lambda e, cnt: (e, 0, 0)),
            pl.BlockSpec((1, MOE_INTER, DIM), lambda e, cnt: (e, 0, 0)),
            pl.BlockSpec((1, DIM, MOE_INTER), lambda e, cnt: (e, 0, 0)),
            pl.BlockSpec((8, 128), lambda e, cnt: (0, 0)),
        ],
        out_specs=pl.BlockSpec((CAPACITY, DIM), lambda e, cnt: (e, 0)),
    )
    return pl.pallas_call(
        _ffn_body,
        grid_spec=grid_spec,
        out_shape=jax.ShapeDtypeStruct((E * CAPACITY, DIM), jnp.float32),
        compiler_params=pltpu.CompilerParams(
            vmem_limit_bytes=60 * 1024 * 1024,
            dimension_semantics=("arbitrary",)),
    )(counts, buf, w1, w3, w2, dep)


def _gather_sc(out_buf, ca1, ca2):
    mesh = plsc.VectorSubcoreMesh(core_axis_name="c", subcore_axis_name="s")

    @functools.partial(
        pl.kernel,
        out_type=[
            jax.ShapeDtypeStruct((T, DIM), jnp.float32),
            jax.ShapeDtypeStruct((T, DIM), jnp.float32),
        ],
        mesh=mesh,
        scratch_types=[
            pltpu.VMEM((HC,), jnp.int32),
            pltpu.VMEM((HC,), jnp.int32),
            pltpu.VMEM((HC,), jnp.int32),
            pltpu.VMEM((HC,), jnp.int32),
            pltpu.VMEM((HC, DIM), jnp.float32),
            pltpu.VMEM((HC, DIM), jnp.float32),
            pltpu.SemaphoreType.DMA,
            pltpu.SemaphoreType.DMA,
        ],
    )
    def k(ob_hbm, ca1_hbm, ca2_hbm, g1_hbm, g2_hbm,
          iA, iB, iC, iD, ra, rb, sg, sw):
        wid = lax.axis_index("s") * SC_CORES + lax.axis_index("c")
        base = wid * TOK_PER_W
        pltpu.sync_copy(ca1_hbm.at[pl.ds(base, HC)], iA)
        pltpu.sync_copy(ca1_hbm.at[pl.ds(base + HC, HC)], iB)
        pltpu.sync_copy(ca2_hbm.at[pl.ds(base, HC)], iC)
        pltpu.sync_copy(ca2_hbm.at[pl.ds(base + HC, HC)], iD)
        gA = pltpu.async_copy(ob_hbm.at[iA], ra, sg)
        gC = pltpu.async_copy(ob_hbm.at[iC], rb, sg)
        gA.wait()
        wA = pltpu.async_copy(ra, g1_hbm.at[pl.ds(base, HC)], sw)
        gC.wait()
        wC = pltpu.async_copy(rb, g2_hbm.at[pl.ds(base, HC)], sw)
        wA.wait()
        gB = pltpu.async_copy(ob_hbm.at[iB], ra, sg)
        wC.wait()
        gD = pltpu.async_copy(ob_hbm.at[iD], rb, sg)
        gB.wait()
        wB = pltpu.async_copy(ra, g1_hbm.at[pl.ds(base + HC, HC)], sw)
        gD.wait()
        wD = pltpu.async_copy(rb, g2_hbm.at[pl.ds(base + HC, HC)], sw)
        wB.wait()
        wD.wait()

    return k(out_buf, ca1, ca2)


def _combine_body(z_ref, g1_ref, g2_ref, wv_ref, y_ref):
    wv = wv_ref[...]
    y_ref[...] = (z_ref[...]
                  + wv[:, 0:1] * g1_ref[...]
                  + wv[:, 1:2] * g2_ref[...])


def _combine_call(z, g1, g2, wv):
    BT = 512
    return pl.pallas_call(
        _combine_body,
        grid=(T // BT,),
        in_specs=[
            pl.BlockSpec((BT, DIM), lambda i: (i, 0)),
            pl.BlockSpec((BT, DIM), lambda i: (i, 0)),
            pl.BlockSpec((BT, DIM), lambda i: (i, 0)),
            pl.BlockSpec((BT, TOPK), lambda i: (i, 0)),
        ],
        out_specs=pl.BlockSpec((BT, DIM), lambda i: (i, 0)),
        out_shape=jax.ShapeDtypeStruct((T, DIM), jnp.float32),
        compiler_params=pltpu.CompilerParams(
            vmem_limit_bytes=60 * 1024 * 1024),
    )(z, g1, g2, wv)


def kernel(x, gate_w, w1, w2, w3, ws1, ws2, ws3):
    da, ca, wv, counts = _gate_call(x, gate_w)
    buf = _dispatch_sc(x, da[:, 0], da[:, 1])
    z_a = _shared_half_a(x, ws1, ws3, ws2)
    out_buf = _ffn_call(counts, buf, w1, w3, w2, z_a)
    g1, g2 = _gather_sc(out_buf, ca[:, 0], ca[:, 1])
    z = _shared_half_b(z_a, x, ws1, ws3, ws2, out_buf)
    return _combine_call(z, g1, g2, wv)


# speedup vs baseline: 1.0537x; 1.0156x over previous
import functools

import jax
import jax.numpy as jnp
from jax import lax
from jax.experimental import pallas as pl
from jax.experimental.pallas import tpu as pltpu
from jax.experimental.pallas import tpu_sc as plsc

DIM = 1024
E = 16
TOPK = 2
N_GROUPS = 4
GROUP_SZ = E // N_GROUPS
TOPK_GROUPS = 2
MOE_INTER = 1024
ROUTE_SCALE = 2.5
T = 2048
CAPACITY = 512

BUF_ROWS = 8192 + 128
TRASH_BASE = 8192

SC_CORES = 2
SC_SUBCORES = 16
SC_WORKERS = SC_CORES * SC_SUBCORES
TOK_PER_W = T // SC_WORKERS


def _gate_body(x_ref, gw_ref, da_ref, ca_ref, wv_ref, cnt_ref):
    x = x_ref[...]
    gw = gw_ref[...]
    logits = lax.dot_general(x, gw, (((1,), (1,)), ((), ())),
                             preferred_element_type=jnp.float32)
    scores = jax.nn.sigmoid(logits)

    lane = lax.broadcasted_iota(jnp.int32, (T, E), 1)
    group_of_lane = lane // GROUP_SZ

    gmax = []
    for g in range(N_GROUPS):
        m = jnp.max(jnp.where(group_of_lane == g, scores, -jnp.inf),
                    axis=1, keepdims=True)
        gmax.append(m)
    sel = []
    for g in range(N_GROUPS):
        rank = jnp.zeros_like(gmax[g])
        for h in range(N_GROUPS):
            if h == g:
                continue
            beats = (gmax[h] > gmax[g]) | ((gmax[h] == gmax[g]) & (h < g))
            rank = rank + beats.astype(jnp.float32)
        sel.append(rank < TOPK_GROUPS)
    group_mask = jnp.zeros((T, E), dtype=jnp.bool_)
    for g in range(N_GROUPS):
        group_mask = group_mask | (sel[g] & (group_of_lane == g))

    ms = jnp.where(group_mask, scores, 0.0)

    BIG = jnp.int32(999)
    v1 = jnp.max(ms, axis=1, keepdims=True)
    i1 = jnp.min(jnp.where(ms == v1, lane, BIG), axis=1, keepdims=True)
    ms2 = jnp.where(lane == i1, -1.0, ms)
    v2 = jnp.max(ms2, axis=1, keepdims=True)
    i2 = jnp.min(jnp.where(ms2 == v2, lane, BIG), axis=1, keepdims=True)

    w1 = jnp.sum(jnp.where(lane == i1, scores, 0.0), axis=1, keepdims=True)
    w2 = jnp.sum(jnp.where(lane == i2, scores, 0.0), axis=1, keepdims=True)
    denom = w1 + w2
    w1n = w1 / denom * ROUTE_SCALE
    w2n = w2 / denom * ROUTE_SCALE

    oh1 = (lane == i1).astype(jnp.float32)
    oh2 = (lane == i2).astype(jnp.float32)
    oh = oh1 + oh2
    CH = 256
    r = lax.broadcasted_iota(jnp.int32, (CH, CH), 0)
    c = lax.broadcasted_iota(jnp.int32, (CH, CH), 1)
    ltri = (c < r).astype(jnp.bfloat16)
    ohb = oh.astype(jnp.bfloat16)
    carry = jnp.zeros((1, E), jnp.float32)
    chunks = []
    for b in range(T // CH):
        chunk = ohb[b * CH:(b + 1) * CH, :]
        chunks.append(jnp.dot(ltri, chunk,
                              preferred_element_type=jnp.float32) + carry)
        carry = carry + jnp.sum(chunk.astype(jnp.float32), axis=0,
                                keepdims=True)
    cum_excl = jnp.concatenate(chunks, axis=0)

    pos1 = jnp.sum(jnp.where(lane == i1, cum_excl, 0.0), axis=1, keepdims=True)
    pos2 = jnp.sum(jnp.where(lane == i2, cum_excl, 0.0), axis=1, keepdims=True)
    pos1 = pos1.astype(jnp.int32)
    pos2 = pos2.astype(jnp.int32)

    valid1 = pos1 < CAPACITY
    valid2 = pos2 < CAPACITY
    pc1 = jnp.minimum(pos1, CAPACITY - 1)
    pc2 = jnp.minimum(pos2, CAPACITY - 1)
    ca1 = i1 * CAPACITY + pc1
    ca2 = i2 * CAPACITY + pc2

    trow = lax.broadcasted_iota(jnp.int32, (T, 1), 0)
    trash = TRASH_BASE + (trow & 7)
    da1 = jnp.where(valid1, ca1, trash)
    da2 = jnp.where(valid2, ca2, trash)

    da_ref[...] = jnp.concatenate([da1, da2], axis=1)
    ca_ref[...] = jnp.concatenate([ca1, ca2], axis=1)
    wv_ref[...] = jnp.concatenate(
        [jnp.where(valid1, w1n, 0.0), jnp.where(valid2, w2n, 0.0)], axis=1)
    cnt_ref[...] = jnp.sum(oh, axis=0, keepdims=True).astype(jnp.int32)


def _gate_call(x, gate_w):
    return pl.pallas_call(
        _gate_body,
        out_shape=[
            jax.ShapeDtypeStruct((T, TOPK), jnp.int32),
            jax.ShapeDtypeStruct((T, TOPK), jnp.int32),
            jax.ShapeDtypeStruct((T, TOPK), jnp.float32),
            jax.ShapeDtypeStruct((1, E), jnp.int32),
        ],
        compiler_params=pltpu.CompilerParams(
            vmem_limit_bytes=60 * 1024 * 1024),
    )(x, gate_w)


HC = TOK_PER_W // 2


def _dispatch_sc(x, da1, da2):
    mesh = plsc.VectorSubcoreMesh(core_axis_name="c", subcore_axis_name="s")

    @functools.partial(
        pl.kernel,
        out_type=jax.ShapeDtypeStruct((BUF_ROWS, DIM), jnp.float32),
        mesh=mesh,
        scratch_types=[
            pltpu.VMEM((HC,), jnp.int32),
            pltpu.VMEM((HC,), jnp.int32),
            pltpu.VMEM((HC,), jnp.int32),
            pltpu.VMEM((HC,), jnp.int32),
            pltpu.VMEM((HC, DIM), jnp.float32),
            pltpu.VMEM((HC, DIM), jnp.float32),
            pltpu.SemaphoreType.DMA,
            pltpu.SemaphoreType.DMA,
            pltpu.SemaphoreType.DMA,
        ],
    )
    def k(x_hbm, da1_hbm, da2_hbm, buf_hbm,
          i1a, i1b, i2a, i2b, ra, rb, sa, sb, ss):
        wid = lax.axis_index("s") * SC_CORES + lax.axis_index("c")
        base = wid * TOK_PER_W
        la = pltpu.async_copy(x_hbm.at[pl.ds(base, HC)], ra, sa)
        lb = pltpu.async_copy(x_hbm.at[pl.ds(base + HC, HC)], rb, sb)
        pltpu.sync_copy(da1_hbm.at[pl.ds(base, HC)], i1a)
        pltpu.sync_copy(da1_hbm.at[pl.ds(base + HC, HC)], i1b)
        pltpu.sync_copy(da2_hbm.at[pl.ds(base, HC)], i2a)
        pltpu.sync_copy(da2_hbm.at[pl.ds(base + HC, HC)], i2b)
        la.wait()
        s1 = pltpu.async_copy(ra, buf_hbm.at[i1a], ss)
        s2 = pltpu.async_copy(ra, buf_hbm.at[i2a], ss)
        lb.wait()
        s3 = pltpu.async_copy(rb, buf_hbm.at[i1b], ss)
        s4 = pltpu.async_copy(rb, buf_hbm.at[i2b], ss)
        s1.wait()
        s2.wait()
        s3.wait()
        s4.wait()

    return k(x, da1, da2)


def _shared_body(x_ref, ws1_ref, ws3_ref, ws2_ref, z_ref):
    xb = x_ref[...].astype(jnp.bfloat16)
    nt = (((1,), (1,)), ((), ()))
    h = lax.dot_general(xb, ws1_ref[...].astype(jnp.bfloat16), nt,
                        preferred_element_type=jnp.float32)
    g = lax.dot_general(xb, ws3_ref[...].astype(jnp.bfloat16), nt,
                        preferred_element_type=jnp.float32)
    a = (jax.nn.silu(h) * g).astype(jnp.bfloat16)
    z_ref[...] = lax.dot_general(a, ws2_ref[...].astype(jnp.bfloat16), nt,
                                 preferred_element_type=jnp.float32)


def _shared_call(x, ws1, ws3, ws2):
    BT = 512
    return pl.pallas_call(
        _shared_body,
        grid=(T // BT,),
        in_specs=[
            pl.BlockSpec((BT, DIM), lambda i: (i, 0)),
            pl.BlockSpec(ws1.shape, lambda i: (0, 0)),
            pl.BlockSpec(ws3.shape, lambda i: (0, 0)),
            pl.BlockSpec(ws2.shape, lambda i: (0, 0)),
        ],
        out_specs=pl.BlockSpec((BT, DIM), lambda i: (i, 0)),
        out_shape=jax.ShapeDtypeStruct((T, DIM), jnp.float32),
        compiler_params=pltpu.CompilerParams(
            vmem_limit_bytes=60 * 1024 * 1024),
    )(x, ws1, ws3, ws2)


FFN_BM = 64


def _ffn_body(cnt_ref, buf_ref, w1_ref, w3_ref, w2_ref, out_ref):
    e = pl.program_id(0)
    c = jnp.minimum(cnt_ref[0, e], CAPACITY)
    nb = (c + FFN_BM - 1) // FFN_BM

    @pl.when(nb > 0)
    def _():
        w1b = w1_ref[0].astype(jnp.bfloat16)
        w3b = w3_ref[0].astype(jnp.bfloat16)
        w2b = w2_ref[0].astype(jnp.bfloat16)
        nt = (((1,), (1,)), ((), ()))

        def make_branch(m_rows):
            def br():
                xb = buf_ref[0:m_rows, :].astype(jnp.bfloat16)
                h = lax.dot_general(xb, w1b, nt,
                                    preferred_element_type=jnp.float32)
                g = lax.dot_general(xb, w3b, nt,
                                    preferred_element_type=jnp.float32)
                a = (jax.nn.silu(h) * g).astype(jnp.bfloat16)
                out_ref[0:m_rows, :] = lax.dot_general(
                    a, w2b, nt, preferred_element_type=jnp.float32)
            return br

        lax.switch(nb - 1,
                   [make_branch(m * FFN_BM)
                    for m in range(1, CAPACITY // FFN_BM + 1)])


def _ffn_call(counts, buf, w1, w3, w2):
    grid_spec = pltpu.PrefetchScalarGridSpec(
        num_scalar_prefetch=1,
        grid=(E,),
        in_specs=[
            pl.BlockSpec((CAPACITY, DIM), lambda e, cnt: (e, 0)),
            pl.BlockSpec((1, MOE_INTER, DIM), lambda e, cnt: (e, 0, 0)),
            pl.BlockSpec((1, MOE_INTER, DIM), lambda e, cnt: (e, 0, 0)),
            pl.BlockSpec((1, DIM, MOE_INTER), lambda e, cnt: (e, 0, 0)),
        ],
        out_specs=pl.BlockSpec((CAPACITY, DIM), lambda e, cnt: (e, 0)),
    )
    return pl.pallas_call(
        _ffn_body,
        grid_spec=grid_spec,
        out_shape=jax.ShapeDtypeStruct((E * CAPACITY, DIM), jnp.float32),
        compiler_params=pltpu.CompilerParams(
            vmem_limit_bytes=60 * 1024 * 1024,
            dimension_semantics=("arbitrary",)),
    )(counts, buf, w1, w3, w2)


def _gather_sc(out_buf, ca1, ca2):
    mesh = plsc.VectorSubcoreMesh(core_axis_name="c", subcore_axis_name="s")

    @functools.partial(
        pl.kernel,
        out_type=[
            jax.ShapeDtypeStruct((T, DIM), jnp.float32),
            jax.ShapeDtypeStruct((T, DIM), jnp.float32),
        ],
        mesh=mesh,
        scratch_types=[
            pltpu.VMEM((HC,), jnp.int32),
            pltpu.VMEM((HC,), jnp.int32),
            pltpu.VMEM((HC,), jnp.int32),
            pltpu.VMEM((HC,), jnp.int32),
            pltpu.VMEM((HC, DIM), jnp.float32),
            pltpu.VMEM((HC, DIM), jnp.float32),
            pltpu.SemaphoreType.DMA,
            pltpu.SemaphoreType.DMA,
        ],
    )
    def k(ob_hbm, ca1_hbm, ca2_hbm, g1_hbm, g2_hbm,
          iA, iB, iC, iD, ra, rb, sg, sw):
        wid = lax.axis_index("s") * SC_CORES + lax.axis_index("c")
        base = wid * TOK_PER_W
        pltpu.sync_copy(ca1_hbm.at[pl.ds(base, HC)], iA)
        pltpu.sync_copy(ca1_hbm.at[pl.ds(base + HC, HC)], iB)
        pltpu.sync_copy(ca2_hbm.at[pl.ds(base, HC)], iC)
        pltpu.sync_copy(ca2_hbm.at[pl.ds(base + HC, HC)], iD)
        gA = pltpu.async_copy(ob_hbm.at[iA], ra, sg)
        gC = pltpu.async_copy(ob_hbm.at[iC], rb, sg)
        gA.wait()
        wA = pltpu.async_copy(ra, g1_hbm.at[pl.ds(base, HC)], sw)
        gC.wait()
        wC = pltpu.async_copy(rb, g2_hbm.at[pl.ds(base, HC)], sw)
        wA.wait()
        gB = pltpu.async_copy(ob_hbm.at[iB], ra, sg)
        wC.wait()
        gD = pltpu.async_copy(ob_hbm.at[iD], rb, sg)
        gB.wait()
        wB = pltpu.async_copy(ra, g1_hbm.at[pl.ds(base + HC, HC)], sw)
        gD.wait()
        wD = pltpu.async_copy(rb, g2_hbm.at[pl.ds(base + HC, HC)], sw)
        wB.wait()
        wD.wait()

    return k(out_buf, ca1, ca2)


def _combine_body(z_ref, g1_ref, g2_ref, wv_ref, y_ref):
    wv = wv_ref[...]
    y_ref[...] = (z_ref[...]
                  + wv[:, 0:1] * g1_ref[...]
                  + wv[:, 1:2] * g2_ref[...])


def _combine_call(z, g1, g2, wv):
    BT = 512
    return pl.pallas_call(
        _combine_body,
        grid=(T // BT,),
        in_specs=[
            pl.BlockSpec((BT, DIM), lambda i: (i, 0)),
            pl.BlockSpec((BT, DIM), lambda i: (i, 0)),
            pl.BlockSpec((BT, DIM), lambda i: (i, 0)),
            pl.BlockSpec((BT, TOPK), lambda i: (i, 0)),
        ],
        out_specs=pl.BlockSpec((BT, DIM), lambda i: (i, 0)),
        out_shape=jax.ShapeDtypeStruct((T, DIM), jnp.float32),
        compiler_params=pltpu.CompilerParams(
            vmem_limit_bytes=60 * 1024 * 1024),
    )(z, g1, g2, wv)


def kernel(x, gate_w, w1, w2, w3, ws1, ws2, ws3):
    da, ca, wv, counts = _gate_call(x, gate_w)
    buf = _dispatch_sc(x, da[:, 0], da[:, 1])
    z = _shared_call(x, ws1, ws3, ws2)
    out_buf = _ffn_call(counts, buf, w1, w3, w2)
    g1, g2 = _gather_sc(out_buf, ca[:, 0], ca[:, 1])
    return _combine_call(z, g1, g2, wv)
